# baseline (speedup 1.0000x reference)
import functools

import jax
import jax.numpy as jnp
from jax import lax
from jax.experimental import pallas as pl
from jax.experimental.pallas import tpu as pltpu
from jax.experimental.pallas import tpu_sc as plsc

N = 10000
E = 320000
DIN = 128
DOUT = 128
DA = 64
NEG_SLOPE = 0.2
SHIFT = 8.0
EPS = 1e-15

NP_ = 10240
NC = 2
NS = 16
L = 16
NW = NC * NS
CH = 80
NCHT = E // CH
NCH = E // NW // CH
NPS = NP_ // NS
DH = DIN // 2
NCH2 = E // NS // CH
CPAD = 8

_f32 = jnp.float32
_i32 = jnp.int32


def _zero_1d(ref, n):
    def body(i, _):
        ref[pl.ds(i * L, L)] = jnp.zeros((L,), _f32)
        return 0
    lax.fori_loop(0, n // L, body, 0)


def _pass1_body(ar, ac, rowi2, coli2, adj2, u_out2, s1p, tp,
                idxr_v, idxc_v, adj_v, u_v, arr0, acr0, arr1, acr1,
                pbuf, pbuf2,
                s1t, tt, red_v, racc_v, sh, sg0, sg1):
    cid = lax.axis_index("c")
    sid = lax.axis_index("s")
    wid = sid * NC + cid
    base = wid * NCH

    _zero_1d(s1t, NP_)
    _zero_1d(tt, NP_)
    pltpu.sync_copy(rowi2.at[pl.ds(base, NCH)], idxr_v)
    pltpu.sync_copy(coli2.at[pl.ds(base, NCH)], idxc_v)
    pltpu.sync_copy(adj2.at[pl.ds(base, NCH)], adj_v)

    def start_g(k, arr, acr, sem):
        pltpu.async_copy(ar.at[idxr_v.at[k]], arr, sem)
        pltpu.async_copy(ac.at[idxc_v.at[k]], acr, sem)

    def wait_g(k, arr, acr, sem):
        pltpu.make_async_copy(ar.at[idxr_v.at[k]], arr, sem).wait()
        pltpu.make_async_copy(ac.at[idxc_v.at[k]], acr, sem).wait()

    iot = lax.iota(_i32, L) * L

    def compute(k, arr, acr):
        def stage(g, pb):
            for e in range(L):
                eg = g * L + e
                p0 = arr[eg, pl.ds(0, L)] * acr[eg, pl.ds(0, L)]
                p1 = arr[eg, pl.ds(L, L)] * acr[eg, pl.ds(L, L)]
                p2 = arr[eg, pl.ds(2 * L, L)] * acr[eg, pl.ds(2 * L, L)]
                p3 = arr[eg, pl.ds(3 * L, L)] * acr[eg, pl.ds(3 * L, L)]
                pb[pl.ds(e * L, L)] = (p0 + p1) + (p2 + p3)

        def finish(g, pb):
            vs = [plsc.load_gather(pb, [iot + j]) for j in range(L)]
            while len(vs) > 1:
                nxt = [vs[i] + vs[i + 1] for i in range(0, len(vs) - 1, 2)]
                if len(vs) % 2:
                    nxt.append(vs[-1])
                vs = nxt
            s = vs[0] * 0.125
            s = jnp.where(s >= 0, s, NEG_SLOPE * s)
            aa = jnp.exp(s - SHIFT)
            gl = pl.ds(g * L, L)
            uu = adj_v[k, gl] * aa
            u_v[k, gl] = uu
            ridx = idxr_v[k, gl]
            plsc.addupdate_scatter(s1t, [ridx], aa)
            plsc.addupdate_scatter(tt, [ridx], uu)

        bufs = (pbuf, pbuf2)
        stage(0, bufs[0])
        for g in range(1, CH // L):
            stage(g, bufs[g % 2])
            finish(g - 1, bufs[(g - 1) % 2])
        finish(CH // L - 1, bufs[(CH // L - 1) % 2])

    start_g(0, arr0, acr0, sg0)

    def pair(q, _):
        k = 2 * q
        start_g(k + 1, arr1, acr1, sg1)
        wait_g(k, arr0, acr0, sg0)
        compute(k, arr0, acr0)
        start_g(k + 2, arr0, acr0, sg0)
        wait_g(k + 1, arr1, acr1, sg1)
        compute(k + 1, arr1, acr1)
        return 0

    lax.fori_loop(0, NCH // 2, pair, 0)
    wait_g(NCH - 1, arr0, acr0, sg0)
    compute(NCH - 1, arr0, acr0)
    pltpu.sync_copy(u_v, u_out2.at[pl.ds(base, NCH)])

    pltpu.sync_copy(s1t, sh.at[sid, 0])
    pltpu.sync_copy(tt, sh.at[sid, 1])
    plsc.subcore_barrier()
    for which in (0, 1):
        _zero_1d(racc_v, NPS)

        def red_j(j, _):
            pltpu.sync_copy(sh.at[j, which, pl.ds(sid * NPS, NPS)], red_v)

            def addv(i, _):
                racc_v[pl.ds(i * L, L)] = (racc_v[pl.ds(i * L, L)]
                                           + red_v[pl.ds(i * L, L)])
                return 0

            lax.fori_loop(0, NPS // L, addv, 0)
            return 0

        lax.fori_loop(0, NS, red_j, 0)
        dst = s1p if which == 0 else tp
        pltpu.sync_copy(racc_v, dst.at[cid, pl.ds(sid * NPS, NPS)])


@jax.jit
def _pass1(ar, ac, rowi2, coli2, adj2):
    mesh = plsc.VectorSubcoreMesh(core_axis_name="c", subcore_axis_name="s")
    return pl.kernel(
        _pass1_body,
        out_type=(
            jax.ShapeDtypeStruct((NCHT, CH), _f32),
            jax.ShapeDtypeStruct((NC, NP_), _f32),
            jax.ShapeDtypeStruct((NC, NP_), _f32),
        ),
        mesh=mesh,
        scratch_types=[
            pltpu.VMEM((NCH, CH), _i32),
            pltpu.VMEM((NCH, CH), _i32),
            pltpu.VMEM((NCH, CH), _f32),
            pltpu.VMEM((NCH, CH), _f32),
            pltpu.VMEM((CH, DA), _f32),
            pltpu.VMEM((CH, DA), _f32),
            pltpu.VMEM((CH, DA), _f32),
            pltpu.VMEM((CH, DA), _f32),
            pltpu.VMEM((L * L,), _f32),
            pltpu.VMEM((L * L,), _f32),
            pltpu.VMEM((NP_,), _f32),
            pltpu.VMEM((NP_,), _f32),
            pltpu.VMEM((NPS,), _f32),
            pltpu.VMEM((NPS,), _f32),
            pltpu.VMEM_SHARED((NS, 2, NP_), _f32),
            pltpu.SemaphoreType.DMA,
            pltpu.SemaphoreType.DMA,
        ],
        compiler_params=pltpu.CompilerParams(needs_layout_passes=False,
                                             use_tc_tiling_on_sc=False),
    )(ar, ac, rowi2, coli2, adj2)


def _pass2_body(xs, rowi2, coli2s, u2, outp,
                idxr_v, idxc_v, uw_v,
                r0, r1, r2, acc_sh,
                sg0, sg1, sg2, ss0, ss1, ss2):
    cid = lax.axis_index("c")
    sid = lax.axis_index("s")
    base = sid * NCH2

    pltpu.sync_copy(rowi2.at[pl.ds(base, NCH2)], idxr_v)
    pltpu.sync_copy(coli2s.at[cid, pl.ds(base, NCH2)], idxc_v)
    pltpu.sync_copy(u2.at[pl.ds(base, NCH2)], uw_v)

    def zrow(i, _):
        for j in range(DH // L):
            r0[i, pl.ds(j * L, L)] = jnp.zeros((L,), _f32)
        return 0

    lax.fori_loop(0, CH, zrow, 0)

    def zcopy(m, _):
        pltpu.sync_copy(r0, acc_sh.at[pl.ds(sid * NPS + m * CH, CH)])
        return 0

    lax.fori_loop(0, NPS // CH, zcopy, 0)
    plsc.subcore_barrier()

    rows = (r0, r1, r2)
    sgs = (sg0, sg1, sg2)
    sss = (ss0, ss1, ss2)

    def start_gather(k, slot):
        pltpu.async_copy(xs.at[idxc_v.at[k]], rows[slot], sgs[slot])

    def wait_gather(k, slot):
        pltpu.make_async_copy(xs.at[idxc_v.at[k]], rows[slot],
                              sgs[slot]).wait()

    def start_scatter(k, slot):
        pltpu.async_copy(rows[slot], acc_sh.at[idxr_v.at[k]], sss[slot],
                         add=True)

    def wait_scatter(k, slot):
        pltpu.make_async_copy(rows[slot], acc_sh.at[idxr_v.at[k]],
                              sss[slot]).wait()

    def mult(k, slot):
        r = rows[slot]
        for g in range(CH // L):
            w16 = uw_v[k, pl.ds(g * L, L)]
            for e in range(L):
                w = w16[e]
                eg = g * L + e
                for j in range(DH // L):
                    r[eg, pl.ds(j * L, L)] = r[eg, pl.ds(j * L, L)] * w

    start_gather(0, 0)
    start_gather(1, 1)
    wait_gather(0, 0)
    mult(0, 0)
    start_scatter(0, 0)
    start_gather(2, 2)
    wait_gather(1, 1)
    mult(1, 1)
    start_scatter(1, 1)
    wait_scatter(0, 0)
    start_gather(3, 0)

    def triple(q, _):
        kq = 3 * q + 2
        for s in range(3):
            k = kq + s
            slot = (2 + s) % 3
            wait_gather(k, slot)
            mult(k, slot)
            start_scatter(k, slot)
            wait_scatter(k - 1, (1 + s) % 3)
            start_gather(k + 2, (1 + s) % 3)
        return 0

    lax.fori_loop(0, (NCH2 - 4) // 3, triple, 0)
    wait_gather(NCH2 - 2, (NCH2 - 2) % 3)
    mult(NCH2 - 2, (NCH2 - 2) % 3)
    start_scatter(NCH2 - 2, (NCH2 - 2) % 3)
    wait_gather(NCH2 - 1, (NCH2 - 1) % 3)
    mult(NCH2 - 1, (NCH2 - 1) % 3)
    start_scatter(NCH2 - 1, (NCH2 - 1) % 3)
    wait_scatter(NCH2 - 3, (NCH2 - 3) % 3)
    wait_scatter(NCH2 - 2, (NCH2 - 2) % 3)
    wait_scatter(NCH2 - 1, (NCH2 - 1) % 3)
    plsc.subcore_barrier()
    pltpu.sync_copy(acc_sh.at[pl.ds(sid * NPS, NPS)],
                    outp.at[cid, pl.ds(sid * NPS, NPS)])


@jax.jit
def _pass2(xs, rowi2, coli2s, u2):
    mesh = plsc.VectorSubcoreMesh(core_axis_name="c", subcore_axis_name="s")
    return pl.kernel(
        _pass2_body,
        out_type=jax.ShapeDtypeStruct((NC, NP_, DH), _f32),
        mesh=mesh,
        scratch_types=[
            pltpu.VMEM((NCH2, CH), _i32),
            pltpu.VMEM((NCH2, CH), _i32),
            pltpu.VMEM((NCH2, CH), _f32),
            pltpu.VMEM((CH, DH), _f32),
            pltpu.VMEM((CH, DH), _f32),
            pltpu.VMEM((CH, DH), _f32),
            pltpu.VMEM_SHARED((NP_, DH), _f32),
            pltpu.SemaphoreType.DMA,
            pltpu.SemaphoreType.DMA,
            pltpu.SemaphoreType.DMA,
            pltpu.SemaphoreType.DMA,
            pltpu.SemaphoreType.DMA,
            pltpu.SemaphoreType.DMA,
        ],
        compiler_params=pltpu.CompilerParams(needs_layout_passes=False,
                                             use_tc_tiling_on_sc=False),
    )(xs, rowi2, coli2s, u2)


BM = 1024


def _proj_body(x_ref, wr_ref, wc_ref, ar_ref, ac_ref):
    xv = x_ref[...]
    ar_ref[...] = jnp.dot(xv, wr_ref[...], preferred_element_type=_f32)
    ac_ref[...] = jnp.dot(xv, wc_ref[...], preferred_element_type=_f32)


@jax.jit
def _proj(xp, wrt, wct):
    return pl.pallas_call(
        _proj_body,
        grid=(NP_ // BM,),
        in_specs=[
            pl.BlockSpec((BM, DIN), lambda i: (i, 0)),
            pl.BlockSpec((DIN, DA), lambda i: (0, 0)),
            pl.BlockSpec((DIN, DA), lambda i: (0, 0)),
        ],
        out_specs=[
            pl.BlockSpec((BM, DA), lambda i: (i, 0)),
            pl.BlockSpec((BM, DA), lambda i: (i, 0)),
        ],
        out_shape=[
            jax.ShapeDtypeStruct((NP_, DA), _f32),
            jax.ShapeDtypeStruct((NP_, DA), _f32),
        ],
    )(xp, wrt, wct)


def _scale_body(s1p_ref, tp_ref, o_ref):
    s = s1p_ref[...]
    t = tp_ref[...]
    s1 = s[0:1, :] + s[1:2, :]
    tt = t[0:1, :] + t[1:2, :]
    d1 = 1.0 / (s1 + EPS)
    d2 = 1.0 / (d1 * tt + EPS)
    o_ref[...] = d1 * d2


@jax.jit
def _scale(s1p, tp):
    return pl.pallas_call(
        _scale_body,
        out_shape=jax.ShapeDtypeStruct((1, NP_), _f32),
    )(s1p, tp)


def _final_body(p0_ref, p1_ref, w0_ref, w1_ref, scl_ref, b_ref, o_ref):
    mm = (jnp.dot(p0_ref[...], w0_ref[...], preferred_element_type=_f32)
          + jnp.dot(p1_ref[...], w1_ref[...], preferred_element_type=_f32))
    o_ref[...] = mm * scl_ref[...] + b_ref[...]


@jax.jit
def _final(p0, p1, wxt0, wxt1, scl_col, b):
    return pl.pallas_call(
        _final_body,
        grid=(NP_ // BM,),
        in_specs=[
            pl.BlockSpec((BM, DH), lambda i: (i, 0)),
            pl.BlockSpec((BM, DH), lambda i: (i, 0)),
            pl.BlockSpec((DH, DOUT), lambda i: (0, 0)),
            pl.BlockSpec((DH, DOUT), lambda i: (0, 0)),
            pl.BlockSpec((BM, 1), lambda i: (i, 0)),
            pl.BlockSpec((1, DOUT), lambda i: (0, 0)),
        ],
        out_specs=pl.BlockSpec((BM, DOUT), lambda i: (i, 0)),
        out_shape=jax.ShapeDtypeStruct((NP_, DOUT), _f32),
    )(p0, p1, wxt0, wxt1, scl_col, b)


def kernel(x, edge_index, adj_values, W_row, W_col, W_x, b_x):
    xp = jnp.pad(x, ((0, NP_ - N), (0, 0)))
    xs = jnp.concatenate([xp[:, :DH], xp[:, DH:]], axis=0)
    rowi2 = edge_index[0].reshape(NCHT, CH)
    coli2 = edge_index[1].reshape(NCHT, CH)
    coli2s = jnp.stack([coli2, coli2 + NP_])
    adj2 = adj_values.reshape(NCHT, CH)
    ar, ac = _proj(xp, W_row.T, W_col.T)
    u2, s1p, tp = _pass1(ar, ac, rowi2, coli2, adj2)
    scl = _scale(s1p, tp)
    outp = _pass2(xs, rowi2, coli2s, u2)
    wxt = W_x.T
    out = _final(outp[0], outp[1], wxt[:DH], wxt[DH:],
                 scl.reshape(NP_, 1), b_x.reshape(1, DOUT))
    return out[:N]


# speedup vs baseline: 1.3327x; 1.3327x over previous
import functools

import jax
import jax.numpy as jnp
from jax import lax
from jax.experimental import pallas as pl
from jax.experimental.pallas import tpu as pltpu
from jax.experimental.pallas import tpu_sc as plsc

N = 10000
E = 320000
DIN = 128
DOUT = 128
DA = 64
NEG_SLOPE = 0.2
SHIFT = 8.0
EPS = 1e-15

NP_ = 10240
NC = 2
NS = 16
L = 16
NW = NC * NS
CH = 80
NCHT = E // CH
NCH = E // NW // CH
NPS = NP_ // NS
DH = DIN // 2
NCH2 = E // NS // CH
CPAD = 8

_f32 = jnp.float32
_bf16 = jnp.bfloat16
_i32 = jnp.int32


def _zero_1d(ref, n):
    def body(i, _):
        ref[pl.ds(i * L, L)] = jnp.zeros((L,), _f32)
        return 0
    lax.fori_loop(0, n // L, body, 0)


def _pass1_body(ar, ac, rowi2, coli2, adj2, u_out2, s1p, tp,
                idxr_v, idxc_v, adj_v, u_v, arr0, acr0, arr1, acr1,
                pbuf, pbuf2,
                s1t, tt, red_v, racc_v, sh, sg0, sg1):
    cid = lax.axis_index("c")
    sid = lax.axis_index("s")
    wid = sid * NC + cid
    base = wid * NCH

    _zero_1d(s1t, NP_)
    _zero_1d(tt, NP_)
    pltpu.sync_copy(rowi2.at[pl.ds(base, NCH)], idxr_v)
    pltpu.sync_copy(coli2.at[pl.ds(base, NCH)], idxc_v)
    pltpu.sync_copy(adj2.at[pl.ds(base, NCH)], adj_v)

    def start_g(k, arr, acr, sem):
        pltpu.async_copy(ar.at[idxr_v.at[k]], arr, sem)
        pltpu.async_copy(ac.at[idxc_v.at[k]], acr, sem)

    def wait_g(k, arr, acr, sem):
        pltpu.make_async_copy(ar.at[idxr_v.at[k]], arr, sem).wait()
        pltpu.make_async_copy(ac.at[idxc_v.at[k]], acr, sem).wait()

    iot = lax.iota(_i32, L) * L

    def compute(k, arr, acr):
        def stage(g, pb):
            for e in range(L):
                eg = g * L + e
                p0 = arr[eg, pl.ds(0, 2 * L)] * acr[eg, pl.ds(0, 2 * L)]
                p1 = (arr[eg, pl.ds(2 * L, 2 * L)]
                      * acr[eg, pl.ds(2 * L, 2 * L)])
                q0, q1 = plsc.unpack(p0, format=plsc.PackFormat.INTERLEAVED)
                q2, q3 = plsc.unpack(p1, format=plsc.PackFormat.INTERLEAVED)
                pb[pl.ds(e * L, L)] = (q0 + q1) + (q2 + q3)

        def finish(g, pb):
            vs = [plsc.load_gather(pb, [iot + j]) for j in range(L)]
            while len(vs) > 1:
                nxt = [vs[i] + vs[i + 1] for i in range(0, len(vs) - 1, 2)]
                if len(vs) % 2:
                    nxt.append(vs[-1])
                vs = nxt
            s = vs[0] * 0.125
            s = jnp.where(s >= 0, s, NEG_SLOPE * s)
            aa = jnp.exp(s - SHIFT)
            gl = pl.ds(g * L, L)
            uu = adj_v[k, gl] * aa
            u_v[k, gl] = uu
            ridx = idxr_v[k, gl]
            plsc.addupdate_scatter(s1t, [ridx], aa)
            plsc.addupdate_scatter(tt, [ridx], uu)

        bufs = (pbuf, pbuf2)
        stage(0, bufs[0])
        for g in range(1, CH // L):
            stage(g, bufs[g % 2])
            finish(g - 1, bufs[(g - 1) % 2])
        finish(CH // L - 1, bufs[(CH // L - 1) % 2])

    start_g(0, arr0, acr0, sg0)

    def pair(q, _):
        k = 2 * q
        start_g(k + 1, arr1, acr1, sg1)
        wait_g(k, arr0, acr0, sg0)
        compute(k, arr0, acr0)
        start_g(k + 2, arr0, acr0, sg0)
        wait_g(k + 1, arr1, acr1, sg1)
        compute(k + 1, arr1, acr1)
        return 0

    lax.fori_loop(0, NCH // 2, pair, 0)
    wait_g(NCH - 1, arr0, acr0, sg0)
    compute(NCH - 1, arr0, acr0)
    pltpu.sync_copy(u_v, u_out2.at[pl.ds(base, NCH)])

    pltpu.sync_copy(s1t, sh.at[sid, 0])
    pltpu.sync_copy(tt, sh.at[sid, 1])
    plsc.subcore_barrier()
    for which in (0, 1):
        _zero_1d(racc_v, NPS)

        def red_j(j, _):
            pltpu.sync_copy(sh.at[j, which, pl.ds(sid * NPS, NPS)], red_v)

            def addv(i, _):
                racc_v[pl.ds(i * L, L)] = (racc_v[pl.ds(i * L, L)]
                                           + red_v[pl.ds(i * L, L)])
                return 0

            lax.fori_loop(0, NPS // L, addv, 0)
            return 0

        lax.fori_loop(0, NS, red_j, 0)
        dst = s1p if which == 0 else tp
        pltpu.sync_copy(racc_v, dst.at[cid, pl.ds(sid * NPS, NPS)])


@jax.jit
def _pass1(ar, ac, rowi2, coli2, adj2):
    mesh = plsc.VectorSubcoreMesh(core_axis_name="c", subcore_axis_name="s")
    return pl.kernel(
        _pass1_body,
        out_type=(
            jax.ShapeDtypeStruct((NCHT, CH), _f32),
            jax.ShapeDtypeStruct((NC, NP_), _f32),
            jax.ShapeDtypeStruct((NC, NP_), _f32),
        ),
        mesh=mesh,
        scratch_types=[
            pltpu.VMEM((NCH, CH), _i32),
            pltpu.VMEM((NCH, CH), _i32),
            pltpu.VMEM((NCH, CH), _f32),
            pltpu.VMEM((NCH, CH), _f32),
            pltpu.VMEM((CH, DA), _bf16),
            pltpu.VMEM((CH, DA), _bf16),
            pltpu.VMEM((CH, DA), _bf16),
            pltpu.VMEM((CH, DA), _bf16),
            pltpu.VMEM((L * L,), _f32),
            pltpu.VMEM((L * L,), _f32),
            pltpu.VMEM((NP_,), _f32),
            pltpu.VMEM((NP_,), _f32),
            pltpu.VMEM((NPS,), _f32),
            pltpu.VMEM((NPS,), _f32),
            pltpu.VMEM_SHARED((NS, 2, NP_), _f32),
            pltpu.SemaphoreType.DMA,
            pltpu.SemaphoreType.DMA,
        ],
        compiler_params=pltpu.CompilerParams(needs_layout_passes=False,
                                             use_tc_tiling_on_sc=False),
    )(ar, ac, rowi2, coli2, adj2)


def _pass2_body(xs, rowi2, coli2s, u2, outp,
                idxr_v, idxc_v, uw_v,
                r0, r1, r2, acc_sh,
                sg0, sg1, sg2, ss0, ss1, ss2):
    cid = lax.axis_index("c")
    sid = lax.axis_index("s")
    base = sid * NCH2

    pltpu.sync_copy(rowi2.at[pl.ds(base, NCH2)], idxr_v)
    pltpu.sync_copy(coli2s.at[cid, pl.ds(base, NCH2)], idxc_v)
    pltpu.sync_copy(u2.at[pl.ds(base, NCH2)], uw_v)

    def zrow(i, _):
        for j in range(DH // L):
            r0[i, pl.ds(j * L, L)] = jnp.zeros((L,), _f32)
        return 0

    lax.fori_loop(0, CH, zrow, 0)

    def zcopy(m, _):
        pltpu.sync_copy(r0, acc_sh.at[pl.ds(sid * NPS + m * CH, CH)])
        return 0

    lax.fori_loop(0, NPS // CH, zcopy, 0)
    plsc.subcore_barrier()

    rows = (r0, r1, r2)
    sgs = (sg0, sg1, sg2)
    sss = (ss0, ss1, ss2)

    def start_gather(k, slot):
        pltpu.async_copy(xs.at[idxc_v.at[k]], rows[slot], sgs[slot])

    def wait_gather(k, slot):
        pltpu.make_async_copy(xs.at[idxc_v.at[k]], rows[slot],
                              sgs[slot]).wait()

    def start_scatter(k, slot):
        pltpu.async_copy(rows[slot], acc_sh.at[idxr_v.at[k]], sss[slot],
                         add=True)

    def wait_scatter(k, slot):
        pltpu.make_async_copy(rows[slot], acc_sh.at[idxr_v.at[k]],
                              sss[slot]).wait()

    def mult(k, slot):
        r = rows[slot]
        for g in range(CH // L):
            w16 = uw_v[k, pl.ds(g * L, L)]
            for e in range(L):
                w = w16[e]
                eg = g * L + e
                for j in range(DH // L):
                    r[eg, pl.ds(j * L, L)] = r[eg, pl.ds(j * L, L)] * w

    start_gather(0, 0)
    start_gather(1, 1)
    wait_gather(0, 0)
    mult(0, 0)
    start_scatter(0, 0)
    start_gather(2, 2)
    wait_gather(1, 1)
    mult(1, 1)
    start_scatter(1, 1)
    wait_scatter(0, 0)
    start_gather(3, 0)

    def triple(q, _):
        kq = 3 * q + 2
        for s in range(3):
            k = kq + s
            slot = (2 + s) % 3
            wait_gather(k, slot)
            mult(k, slot)
            start_scatter(k, slot)
            wait_scatter(k - 1, (1 + s) % 3)
            start_gather(k + 2, (1 + s) % 3)
        return 0

    lax.fori_loop(0, (NCH2 - 4) // 3, triple, 0)
    wait_gather(NCH2 - 2, (NCH2 - 2) % 3)
    mult(NCH2 - 2, (NCH2 - 2) % 3)
    start_scatter(NCH2 - 2, (NCH2 - 2) % 3)
    wait_gather(NCH2 - 1, (NCH2 - 1) % 3)
    mult(NCH2 - 1, (NCH2 - 1) % 3)
    start_scatter(NCH2 - 1, (NCH2 - 1) % 3)
    wait_scatter(NCH2 - 3, (NCH2 - 3) % 3)
    wait_scatter(NCH2 - 2, (NCH2 - 2) % 3)
    wait_scatter(NCH2 - 1, (NCH2 - 1) % 3)
    plsc.subcore_barrier()
    pltpu.sync_copy(acc_sh.at[pl.ds(sid * NPS, NPS)],
                    outp.at[cid, pl.ds(sid * NPS, NPS)])


@jax.jit
def _pass2(xs, rowi2, coli2s, u2):
    mesh = plsc.VectorSubcoreMesh(core_axis_name="c", subcore_axis_name="s")
    return pl.kernel(
        _pass2_body,
        out_type=jax.ShapeDtypeStruct((NC, NP_, DH), _f32),
        mesh=mesh,
        scratch_types=[
            pltpu.VMEM((NCH2, CH), _i32),
            pltpu.VMEM((NCH2, CH), _i32),
            pltpu.VMEM((NCH2, CH), _f32),
            pltpu.VMEM((CH, DH), _f32),
            pltpu.VMEM((CH, DH), _f32),
            pltpu.VMEM((CH, DH), _f32),
            pltpu.VMEM_SHARED((NP_, DH), _f32),
            pltpu.SemaphoreType.DMA,
            pltpu.SemaphoreType.DMA,
            pltpu.SemaphoreType.DMA,
            pltpu.SemaphoreType.DMA,
            pltpu.SemaphoreType.DMA,
            pltpu.SemaphoreType.DMA,
        ],
        compiler_params=pltpu.CompilerParams(needs_layout_passes=False,
                                             use_tc_tiling_on_sc=False),
    )(xs, rowi2, coli2s, u2)


BM = 1024


def _proj_body(x_ref, wr_ref, wc_ref, ar_ref, ac_ref):
    xv = x_ref[...]
    ar_ref[...] = jnp.dot(xv, wr_ref[...],
                          preferred_element_type=_f32).astype(_bf16)
    ac_ref[...] = jnp.dot(xv, wc_ref[...],
                          preferred_element_type=_f32).astype(_bf16)


@jax.jit
def _proj(xp, wrt, wct):
    return pl.pallas_call(
        _proj_body,
        grid=(NP_ // BM,),
        in_specs=[
            pl.BlockSpec((BM, DIN), lambda i: (i, 0)),
            pl.BlockSpec((DIN, DA), lambda i: (0, 0)),
            pl.BlockSpec((DIN, DA), lambda i: (0, 0)),
        ],
        out_specs=[
            pl.BlockSpec((BM, DA), lambda i: (i, 0)),
            pl.BlockSpec((BM, DA), lambda i: (i, 0)),
        ],
        out_shape=[
            jax.ShapeDtypeStruct((NP_, DA), _bf16),
            jax.ShapeDtypeStruct((NP_, DA), _bf16),
        ],
    )(xp, wrt, wct)


def _scale_body(s1p_ref, tp_ref, o_ref):
    s = s1p_ref[...]
    t = tp_ref[...]
    s1 = s[0:1, :] + s[1:2, :]
    tt = t[0:1, :] + t[1:2, :]
    d1 = 1.0 / (s1 + EPS)
    d2 = 1.0 / (d1 * tt + EPS)
    o_ref[...] = d1 * d2


@jax.jit
def _scale(s1p, tp):
    return pl.pallas_call(
        _scale_body,
        out_shape=jax.ShapeDtypeStruct((1, NP_), _f32),
    )(s1p, tp)


def _final_body(p0_ref, p1_ref, w0_ref, w1_ref, scl_ref, b_ref, o_ref):
    mm = (jnp.dot(p0_ref[...], w0_ref[...], preferred_element_type=_f32)
          + jnp.dot(p1_ref[...], w1_ref[...], preferred_element_type=_f32))
    o_ref[...] = mm * scl_ref[...] + b_ref[...]


@jax.jit
def _final(p0, p1, wxt0, wxt1, scl_col, b):
    return pl.pallas_call(
        _final_body,
        grid=(NP_ // BM,),
        in_specs=[
            pl.BlockSpec((BM, DH), lambda i: (i, 0)),
            pl.BlockSpec((BM, DH), lambda i: (i, 0)),
            pl.BlockSpec((DH, DOUT), lambda i: (0, 0)),
            pl.BlockSpec((DH, DOUT), lambda i: (0, 0)),
            pl.BlockSpec((BM, 1), lambda i: (i, 0)),
            pl.BlockSpec((1, DOUT), lambda i: (0, 0)),
        ],
        out_specs=pl.BlockSpec((BM, DOUT), lambda i: (i, 0)),
        out_shape=jax.ShapeDtypeStruct((NP_, DOUT), _f32),
    )(p0, p1, wxt0, wxt1, scl_col, b)


def kernel(x, edge_index, adj_values, W_row, W_col, W_x, b_x):
    xp = jnp.pad(x, ((0, NP_ - N), (0, 0)))
    xs = jnp.concatenate([xp[:, :DH], xp[:, DH:]], axis=0)
    rowi2 = edge_index[0].reshape(NCHT, CH)
    coli2 = edge_index[1].reshape(NCHT, CH)
    coli2s = jnp.stack([coli2, coli2 + NP_])
    adj2 = adj_values.reshape(NCHT, CH)
    ar, ac = _proj(xp, W_row.T, W_col.T)
    u2, s1p, tp = _pass1(ar, ac, rowi2, coli2, adj2)
    scl = _scale(s1p, tp)
    outp = _pass2(xs, rowi2, coli2s, u2)
    wxt = W_x.T
    out = _final(outp[0], outp[1], wxt[:DH], wxt[DH:],
                 scl.reshape(NP_, 1), b_x.reshape(1, DOUT))
    return out[:N]


# speedup vs baseline: 1.3335x; 1.0006x over previous
import functools

import jax
import jax.numpy as jnp
from jax import lax
from jax.experimental import pallas as pl
from jax.experimental.pallas import tpu as pltpu
from jax.experimental.pallas import tpu_sc as plsc

N = 10000
E = 320000
DIN = 128
DOUT = 128
DA = 64
NEG_SLOPE = 0.2
SHIFT = 8.0
EPS = 1e-15

NP_ = 10240
NC = 2
NS = 16
L = 16
NW = NC * NS
CH = 80
NCHT = E // CH
NCH = E // NW // CH
NPS = NP_ // NS
DH = DIN // 2
NCH2 = E // NS // CH

_f32 = jnp.float32
_bf16 = jnp.bfloat16
_i32 = jnp.int32


def _zero_1d(ref, n):
    def body(i, _):
        ref[pl.ds(i * L, L)] = jnp.zeros((L,), _f32)
        return 0
    lax.fori_loop(0, n // L, body, 0)


def _pass1_body(ar, ac, rowi2, coli2, adj2, u_out2, s1p, tp,
                idxr_v, idxc_v, adj_v, u_v, arr0, acr0, arr1, acr1,
                pbuf, pbuf2,
                s1t, tt, red_v, racc_v, sh, sg0, sg1):
    cid = lax.axis_index("c")
    sid = lax.axis_index("s")
    wid = sid * NC + cid
    base = wid * NCH

    _zero_1d(s1t, NP_)
    _zero_1d(tt, NP_)
    pltpu.sync_copy(rowi2.at[pl.ds(base, NCH)], idxr_v)
    pltpu.sync_copy(coli2.at[pl.ds(base, NCH)], idxc_v)
    pltpu.sync_copy(adj2.at[pl.ds(base, NCH)], adj_v)

    def start_g(k, arr, acr, sem):
        pltpu.async_copy(ar.at[idxr_v.at[k]], arr, sem)
        pltpu.async_copy(ac.at[idxc_v.at[k]], acr, sem)

    def wait_g(k, arr, acr, sem):
        pltpu.make_async_copy(ar.at[idxr_v.at[k]], arr, sem).wait()
        pltpu.make_async_copy(ac.at[idxc_v.at[k]], acr, sem).wait()

    iot = lax.iota(_i32, L) * L

    def compute(k, arr, acr):
        def stage(g, pb):
            for e in range(L):
                eg = g * L + e
                p0 = arr[eg, pl.ds(0, 2 * L)] * acr[eg, pl.ds(0, 2 * L)]
                p1 = (arr[eg, pl.ds(2 * L, 2 * L)]
                      * acr[eg, pl.ds(2 * L, 2 * L)])
                q0, q1 = plsc.unpack(p0, format=plsc.PackFormat.INTERLEAVED)
                q2, q3 = plsc.unpack(p1, format=plsc.PackFormat.INTERLEAVED)
                pb[pl.ds(e * L, L)] = (q0 + q1) + (q2 + q3)

        def finish(g, pb):
            vs = [plsc.load_gather(pb, [iot + j]) for j in range(L)]
            while len(vs) > 1:
                nxt = [vs[i] + vs[i + 1] for i in range(0, len(vs) - 1, 2)]
                if len(vs) % 2:
                    nxt.append(vs[-1])
                vs = nxt
            s = vs[0] * 0.125
            s = jnp.where(s >= 0, s, NEG_SLOPE * s)
            aa = jnp.exp(s - SHIFT)
            gl = pl.ds(g * L, L)
            uu = adj_v[k, gl] * aa
            u_v[k, gl] = uu
            ridx = idxr_v[k, gl]
            plsc.addupdate_scatter(s1t, [ridx], aa)
            plsc.addupdate_scatter(tt, [ridx], uu)

        bufs = (pbuf, pbuf2)
        stage(0, bufs[0])
        for g in range(1, CH // L):
            stage(g, bufs[g % 2])
            finish(g - 1, bufs[(g - 1) % 2])
        finish(CH // L - 1, bufs[(CH // L - 1) % 2])

    start_g(0, arr0, acr0, sg0)

    def pair(q, _):
        k = 2 * q
        start_g(k + 1, arr1, acr1, sg1)
        wait_g(k, arr0, acr0, sg0)
        compute(k, arr0, acr0)
        start_g(k + 2, arr0, acr0, sg0)
        wait_g(k + 1, arr1, acr1, sg1)
        compute(k + 1, arr1, acr1)
        return 0

    lax.fori_loop(0, NCH // 2, pair, 0)
    wait_g(NCH - 1, arr0, acr0, sg0)
    compute(NCH - 1, arr0, acr0)
    pltpu.sync_copy(u_v, u_out2.at[pl.ds(base, NCH)])

    pltpu.sync_copy(s1t, sh.at[sid, 0])
    pltpu.sync_copy(tt, sh.at[sid, 1])
    plsc.subcore_barrier()
    for which in (0, 1):
        _zero_1d(racc_v, NPS)

        def red_j(j, _):
            pltpu.sync_copy(sh.at[j, which, pl.ds(sid * NPS, NPS)], red_v)

            def addv(i, _):
                racc_v[pl.ds(i * L, L)] = (racc_v[pl.ds(i * L, L)]
                                           + red_v[pl.ds(i * L, L)])
                return 0

            lax.fori_loop(0, NPS // L, addv, 0)
            return 0

        lax.fori_loop(0, NS, red_j, 0)
        dst = s1p if which == 0 else tp
        pltpu.sync_copy(racc_v, dst.at[cid, pl.ds(sid * NPS, NPS)])


@jax.jit
def _pass1(ar, ac, rowi2, coli2, adj2):
    mesh = plsc.VectorSubcoreMesh(core_axis_name="c", subcore_axis_name="s")
    return pl.kernel(
        _pass1_body,
        out_type=(
            jax.ShapeDtypeStruct((NCHT, CH), _f32),
            jax.ShapeDtypeStruct((NC, NP_), _f32),
            jax.ShapeDtypeStruct((NC, NP_), _f32),
        ),
        mesh=mesh,
        scratch_types=[
            pltpu.VMEM((NCH, CH), _i32),
            pltpu.VMEM((NCH, CH), _i32),
            pltpu.VMEM((NCH, CH), _f32),
            pltpu.VMEM((NCH, CH), _f32),
            pltpu.VMEM((CH, DA), _bf16),
            pltpu.VMEM((CH, DA), _bf16),
            pltpu.VMEM((CH, DA), _bf16),
            pltpu.VMEM((CH, DA), _bf16),
            pltpu.VMEM((L * L,), _f32),
            pltpu.VMEM((L * L,), _f32),
            pltpu.VMEM((NP_,), _f32),
            pltpu.VMEM((NP_,), _f32),
            pltpu.VMEM((NPS,), _f32),
            pltpu.VMEM((NPS,), _f32),
            pltpu.VMEM_SHARED((NS, 2, NP_), _f32),
            pltpu.SemaphoreType.DMA,
            pltpu.SemaphoreType.DMA,
        ],
        compiler_params=pltpu.CompilerParams(needs_layout_passes=False,
                                             use_tc_tiling_on_sc=False),
    )(ar, ac, rowi2, coli2, adj2)


def _pass2_body(xs, rowi2, coli2s, u2, outp,
                idxr_v, idxc_v, uw_v,
                r0, r1, r2, acc_sh,
                sg0, sg1, sg2, ss0, ss1, ss2):
    cid = lax.axis_index("c")
    sid = lax.axis_index("s")
    base = sid * NCH2

    pltpu.sync_copy(rowi2.at[pl.ds(base, NCH2)], idxr_v)
    pltpu.sync_copy(coli2s.at[cid, pl.ds(base, NCH2)], idxc_v)
    pltpu.sync_copy(u2.at[pl.ds(base, NCH2)], uw_v)

    def zrow(i, _):
        for j in range(DH // L):
            r0[i, pl.ds(j * L, L)] = jnp.zeros((L,), _f32)
        return 0

    lax.fori_loop(0, CH, zrow, 0)

    def zcopy(m, _):
        pltpu.sync_copy(r0, acc_sh.at[pl.ds(sid * NPS + m * CH, CH)])
        return 0

    lax.fori_loop(0, NPS // CH, zcopy, 0)
    plsc.subcore_barrier()

    rows = (r0, r1, r2)
    sgs = (sg0, sg1, sg2)
    sss = (ss0, ss1, ss2)

    def start_gather(k, slot):
        pltpu.async_copy(xs.at[idxc_v.at[k]], rows[slot], sgs[slot])

    def wait_gather(k, slot):
        pltpu.make_async_copy(xs.at[idxc_v.at[k]], rows[slot],
                              sgs[slot]).wait()

    def start_scatter(k, slot):
        pltpu.async_copy(rows[slot], acc_sh.at[idxr_v.at[k]], sss[slot],
                         add=True)

    def wait_scatter(k, slot):
        pltpu.make_async_copy(rows[slot], acc_sh.at[idxr_v.at[k]],
                              sss[slot]).wait()

    def mult(k, slot):
        r = rows[slot]
        for g in range(CH // L):
            w16 = uw_v[k, pl.ds(g * L, L)]
            for e in range(L):
                w = w16[e]
                eg = g * L + e
                for j in range(DH // L):
                    r[eg, pl.ds(j * L, L)] = r[eg, pl.ds(j * L, L)] * w

    start_gather(0, 0)
    start_gather(1, 1)
    wait_gather(0, 0)
    mult(0, 0)
    start_scatter(0, 0)
    start_gather(2, 2)
    wait_gather(1, 1)
    mult(1, 1)
    start_scatter(1, 1)
    wait_scatter(0, 0)
    start_gather(3, 0)

    def triple(q, _):
        kq = 3 * q + 2
        for s in range(3):
            k = kq + s
            slot = (2 + s) % 3
            wait_gather(k, slot)
            mult(k, slot)
            start_scatter(k, slot)
            wait_scatter(k - 1, (1 + s) % 3)
            start_gather(k + 2, (1 + s) % 3)
        return 0

    lax.fori_loop(0, (NCH2 - 4) // 3, triple, 0)
    wait_gather(NCH2 - 2, (NCH2 - 2) % 3)
    mult(NCH2 - 2, (NCH2 - 2) % 3)
    start_scatter(NCH2 - 2, (NCH2 - 2) % 3)
    wait_gather(NCH2 - 1, (NCH2 - 1) % 3)
    mult(NCH2 - 1, (NCH2 - 1) % 3)
    start_scatter(NCH2 - 1, (NCH2 - 1) % 3)
    wait_scatter(NCH2 - 3, (NCH2 - 3) % 3)
    wait_scatter(NCH2 - 2, (NCH2 - 2) % 3)
    wait_scatter(NCH2 - 1, (NCH2 - 1) % 3)
    plsc.subcore_barrier()
    pltpu.sync_copy(acc_sh.at[pl.ds(sid * NPS, NPS)],
                    outp.at[cid, pl.ds(sid * NPS, NPS)])


@jax.jit
def _pass2(xs, rowi2, coli2s, u2):
    mesh = plsc.VectorSubcoreMesh(core_axis_name="c", subcore_axis_name="s")
    return pl.kernel(
        _pass2_body,
        out_type=jax.ShapeDtypeStruct((NC, NP_, DH), _f32),
        mesh=mesh,
        scratch_types=[
            pltpu.VMEM((NCH2, CH), _i32),
            pltpu.VMEM((NCH2, CH), _i32),
            pltpu.VMEM((NCH2, CH), _f32),
            pltpu.VMEM((CH, DH), _f32),
            pltpu.VMEM((CH, DH), _f32),
            pltpu.VMEM((CH, DH), _f32),
            pltpu.VMEM_SHARED((NP_, DH), _f32),
            pltpu.SemaphoreType.DMA,
            pltpu.SemaphoreType.DMA,
            pltpu.SemaphoreType.DMA,
            pltpu.SemaphoreType.DMA,
            pltpu.SemaphoreType.DMA,
            pltpu.SemaphoreType.DMA,
        ],
        compiler_params=pltpu.CompilerParams(needs_layout_passes=False,
                                             use_tc_tiling_on_sc=False),
    )(xs, rowi2, coli2s, u2)


BM = 1024


def _proj_body(x_ref, wr_ref, wc_ref, ar_ref, ac_ref):
    xv = x_ref[...]
    ar_ref[...] = jnp.dot(xv, wr_ref[...],
                          preferred_element_type=_f32).astype(_bf16)
    ac_ref[...] = jnp.dot(xv, wc_ref[...],
                          preferred_element_type=_f32).astype(_bf16)


@jax.jit
def _proj(xp, wrt, wct):
    return pl.pallas_call(
        _proj_body,
        grid=(NP_ // BM,),
        in_specs=[
            pl.BlockSpec((BM, DIN), lambda i: (i, 0)),
            pl.BlockSpec((DIN, DA), lambda i: (0, 0)),
            pl.BlockSpec((DIN, DA), lambda i: (0, 0)),
        ],
        out_specs=[
            pl.BlockSpec((BM, DA), lambda i: (i, 0)),
            pl.BlockSpec((BM, DA), lambda i: (i, 0)),
        ],
        out_shape=[
            jax.ShapeDtypeStruct((NP_, DA), _bf16),
            jax.ShapeDtypeStruct((NP_, DA), _bf16),
        ],
    )(xp, wrt, wct)


def _scale_body(s1p_ref, tp_ref, o_ref):
    s = s1p_ref[...]
    t = tp_ref[...]
    s1 = s[0:1, :] + s[1:2, :]
    tt = t[0:1, :] + t[1:2, :]
    d1 = 1.0 / (s1 + EPS)
    d2 = 1.0 / (d1 * tt + EPS)
    o_ref[...] = d1 * d2


@jax.jit
def _scale(s1p, tp):
    return pl.pallas_call(
        _scale_body,
        out_shape=jax.ShapeDtypeStruct((1, NP_), _f32),
    )(s1p, tp)


def _final_body(p0_ref, p1_ref, w0_ref, w1_ref, scl_ref, b_ref, o_ref):
    mm = (jnp.dot(p0_ref[...], w0_ref[...], preferred_element_type=_f32)
          + jnp.dot(p1_ref[...], w1_ref[...], preferred_element_type=_f32))
    o_ref[...] = mm * scl_ref[...] + b_ref[...]


@jax.jit
def _final(p0, p1, wxt0, wxt1, scl_col, b):
    return pl.pallas_call(
        _final_body,
        grid=(NP_ // BM,),
        in_specs=[
            pl.BlockSpec((BM, DH), lambda i: (i, 0)),
            pl.BlockSpec((BM, DH), lambda i: (i, 0)),
            pl.BlockSpec((DH, DOUT), lambda i: (0, 0)),
            pl.BlockSpec((DH, DOUT), lambda i: (0, 0)),
            pl.BlockSpec((BM, 1), lambda i: (i, 0)),
            pl.BlockSpec((1, DOUT), lambda i: (0, 0)),
        ],
        out_specs=pl.BlockSpec((BM, DOUT), lambda i: (i, 0)),
        out_shape=jax.ShapeDtypeStruct((NP_, DOUT), _f32),
    )(p0, p1, wxt0, wxt1, scl_col, b)


def kernel(x, edge_index, adj_values, W_row, W_col, W_x, b_x):
    xp = jnp.pad(x, ((0, NP_ - N), (0, 0)))
    xs = jnp.concatenate([xp[:, :DH], xp[:, DH:]], axis=0)
    rowi2 = edge_index[0].reshape(NCHT, CH)
    coli2 = edge_index[1].reshape(NCHT, CH)
    coli2s = jnp.stack([coli2, coli2 + NP_])
    adj2 = adj_values.reshape(NCHT, CH)
    ar, ac = _proj(xp, W_row.T, W_col.T)
    u2, s1p, tp = _pass1(ar, ac, rowi2, coli2, adj2)
    scl = _scale(s1p, tp)
    outp = _pass2(xs, rowi2, coli2s, u2)
    wxt = W_x.T
    out = _final(outp[0], outp[1], wxt[:DH], wxt[DH:],
                 scl.reshape(NP_, 1), b_x.reshape(1, DOUT))
    return out[:N]


# speedup vs baseline: 1.3438x; 1.0077x over previous
import functools

import jax
import jax.numpy as jnp
from jax import lax
from jax.experimental import pallas as pl
from jax.experimental.pallas import tpu as pltpu
from jax.experimental.pallas import tpu_sc as plsc

N = 10000
E = 320000
DIN = 128
DOUT = 128
DA = 64
NEG_SLOPE = 0.2
SHIFT = 8.0
EPS = 1e-15

NP_ = 10240
NC = 2
NS = 16
L = 16
NW = NC * NS
CH = 80
NCHT = E // CH
NCH = E // NW // CH
NPS = NP_ // NS
DH = DIN // 2
NCH2 = E // NS // CH

_f32 = jnp.float32
_bf16 = jnp.bfloat16
_i32 = jnp.int32


def _zero_1d(ref, n):
    def body(i, _):
        ref[pl.ds(i * L, L)] = jnp.zeros((L,), _f32)
        return 0
    lax.fori_loop(0, n // L, body, 0)


def _pass1_body(ar, ac, rowi2, coli2, adj2, u_out2, s1p, tp,
                idxr_v, idxc_v, adj_v, u_v, arr0, acr0, arr1, acr1,
                pbuf, pbuf2,
                s1t, tt, red_v, racc_v, sh, sg0, sg1):
    cid = lax.axis_index("c")
    sid = lax.axis_index("s")
    wid = sid * NC + cid
    base = wid * NCH

    _zero_1d(s1t, NP_)
    _zero_1d(tt, NP_)
    pltpu.sync_copy(rowi2.at[pl.ds(base, NCH)], idxr_v)
    pltpu.sync_copy(coli2.at[pl.ds(base, NCH)], idxc_v)
    pltpu.sync_copy(adj2.at[pl.ds(base, NCH)], adj_v)

    def start_g(k, arr, acr, sem):
        pltpu.async_copy(ar.at[idxr_v.at[k]], arr, sem)
        pltpu.async_copy(ac.at[idxc_v.at[k]], acr, sem)

    def wait_g(k, arr, acr, sem):
        pltpu.make_async_copy(ar.at[idxr_v.at[k]], arr, sem).wait()
        pltpu.make_async_copy(ac.at[idxc_v.at[k]], acr, sem).wait()

    iot = lax.iota(_i32, L) * L

    def compute(k, arr, acr):
        def stage(g, pb):
            for e in range(L):
                eg = g * L + e
                p0 = arr[eg, pl.ds(0, 2 * L)] * acr[eg, pl.ds(0, 2 * L)]
                p1 = (arr[eg, pl.ds(2 * L, 2 * L)]
                      * acr[eg, pl.ds(2 * L, 2 * L)])
                q0, q1 = plsc.unpack(p0, format=plsc.PackFormat.INTERLEAVED)
                q2, q3 = plsc.unpack(p1, format=plsc.PackFormat.INTERLEAVED)
                pb[pl.ds(e * L, L)] = (q0 + q1) + (q2 + q3)

        def finish(g, pb):
            vs = [plsc.load_gather(pb, [iot + j]) for j in range(L)]
            while len(vs) > 1:
                nxt = [vs[i] + vs[i + 1] for i in range(0, len(vs) - 1, 2)]
                if len(vs) % 2:
                    nxt.append(vs[-1])
                vs = nxt
            s = vs[0] * 0.125
            s = jnp.where(s >= 0, s, NEG_SLOPE * s)
            aa = jnp.exp(s - SHIFT)
            gl = pl.ds(g * L, L)
            uu = adj_v[k, gl] * aa
            u_v[k, gl] = uu
            ridx = idxr_v[k, gl]
            plsc.addupdate_scatter(s1t, [ridx], aa)
            plsc.addupdate_scatter(tt, [ridx], uu)

        bufs = (pbuf, pbuf2)
        stage(0, bufs[0])
        for g in range(1, CH // L):
            stage(g, bufs[g % 2])
            finish(g - 1, bufs[(g - 1) % 2])
        finish(CH // L - 1, bufs[(CH // L - 1) % 2])

    start_g(0, arr0, acr0, sg0)

    def pair(q, _):
        k = 2 * q
        start_g(k + 1, arr1, acr1, sg1)
        wait_g(k, arr0, acr0, sg0)
        compute(k, arr0, acr0)
        start_g(k + 2, arr0, acr0, sg0)
        wait_g(k + 1, arr1, acr1, sg1)
        compute(k + 1, arr1, acr1)
        return 0

    lax.fori_loop(0, NCH // 2, pair, 0)
    wait_g(NCH - 1, arr0, acr0, sg0)
    compute(NCH - 1, arr0, acr0)
    pltpu.sync_copy(u_v, u_out2.at[pl.ds(base, NCH)])

    pltpu.sync_copy(s1t, sh.at[sid, 0])
    pltpu.sync_copy(tt, sh.at[sid, 1])
    plsc.subcore_barrier()
    for which in (0, 1):
        _zero_1d(racc_v, NPS)

        def red_j(j, _):
            pltpu.sync_copy(sh.at[j, which, pl.ds(sid * NPS, NPS)], red_v)

            def addv(i, _):
                racc_v[pl.ds(i * L, L)] = (racc_v[pl.ds(i * L, L)]
                                           + red_v[pl.ds(i * L, L)])
                return 0

            lax.fori_loop(0, NPS // L, addv, 0)
            return 0

        lax.fori_loop(0, NS, red_j, 0)
        dst = s1p if which == 0 else tp
        pltpu.sync_copy(racc_v, dst.at[cid, pl.ds(sid * NPS, NPS)])


@jax.jit
def _pass1(ar, ac, rowi2, coli2, adj2):
    mesh = plsc.VectorSubcoreMesh(core_axis_name="c", subcore_axis_name="s")
    return pl.kernel(
        _pass1_body,
        out_type=(
            jax.ShapeDtypeStruct((NCHT, CH), _f32),
            jax.ShapeDtypeStruct((NC, NP_), _f32),
            jax.ShapeDtypeStruct((NC, NP_), _f32),
        ),
        mesh=mesh,
        scratch_types=[
            pltpu.VMEM((NCH, CH), _i32),
            pltpu.VMEM((NCH, CH), _i32),
            pltpu.VMEM((NCH, CH), _f32),
            pltpu.VMEM((NCH, CH), _f32),
            pltpu.VMEM((CH, DA), _bf16),
            pltpu.VMEM((CH, DA), _bf16),
            pltpu.VMEM((CH, DA), _bf16),
            pltpu.VMEM((CH, DA), _bf16),
            pltpu.VMEM((L * L,), _f32),
            pltpu.VMEM((L * L,), _f32),
            pltpu.VMEM((NP_,), _f32),
            pltpu.VMEM((NP_,), _f32),
            pltpu.VMEM((NPS,), _f32),
            pltpu.VMEM((NPS,), _f32),
            pltpu.VMEM_SHARED((NS, 2, NP_), _f32),
            pltpu.SemaphoreType.DMA,
            pltpu.SemaphoreType.DMA,
        ],
        compiler_params=pltpu.CompilerParams(needs_layout_passes=False,
                                             use_tc_tiling_on_sc=False),
    )(ar, ac, rowi2, coli2, adj2)


def _pass2_body(xs, rowi2, coli2s, u2, outp,
                idxr_v, idxc_v, uw_v,
                r0, r1, r2, acc_sh,
                sg0, sg1, sg2, ss0, ss1, ss2):
    cid = lax.axis_index("c")
    sid = lax.axis_index("s")
    base = sid * NCH2

    pltpu.sync_copy(rowi2.at[pl.ds(base, NCH2)], idxr_v)
    pltpu.sync_copy(coli2s.at[cid, pl.ds(base, NCH2)], idxc_v)
    pltpu.sync_copy(u2.at[pl.ds(base, NCH2)], uw_v)

    def zrow(i, _):
        for j in range(DH // L):
            r0[i, pl.ds(j * L, L)] = jnp.zeros((L,), _f32)
        return 0

    lax.fori_loop(0, CH, zrow, 0)

    def zcopy(m, _):
        pltpu.sync_copy(r0, acc_sh.at[pl.ds(sid * NPS + m * CH, CH)])
        return 0

    lax.fori_loop(0, NPS // CH, zcopy, 0)
    plsc.subcore_barrier()

    rows = (r0, r1, r2)
    sgs = (sg0, sg1, sg2)
    sss = (ss0, ss1, ss2)

    def start_gather(k, slot):
        pltpu.async_copy(xs.at[idxc_v.at[k]], rows[slot], sgs[slot])

    def wait_gather(k, slot):
        pltpu.make_async_copy(xs.at[idxc_v.at[k]], rows[slot],
                              sgs[slot]).wait()

    def start_scatter(k, slot):
        pltpu.async_copy(rows[slot], acc_sh.at[idxr_v.at[k]], sss[slot],
                         add=True)

    def wait_scatter(k, slot):
        pltpu.make_async_copy(rows[slot], acc_sh.at[idxr_v.at[k]],
                              sss[slot]).wait()

    def mult(k, slot):
        r = rows[slot]
        for g in range(CH // L):
            w16 = uw_v[k, pl.ds(g * L, L)]
            for e in range(L):
                w = w16[e]
                eg = g * L + e
                for j in range(DH // L):
                    r[eg, pl.ds(j * L, L)] = r[eg, pl.ds(j * L, L)] * w

    start_gather(0, 0)
    start_gather(1, 1)
    wait_gather(0, 0)
    mult(0, 0)
    start_scatter(0, 0)
    start_gather(2, 2)
    wait_gather(1, 1)
    mult(1, 1)
    start_scatter(1, 1)
    wait_scatter(0, 0)
    start_gather(3, 0)

    def triple(q, _):
        kq = 3 * q + 2
        for s in range(3):
            k = kq + s
            slot = (2 + s) % 3
            wait_gather(k, slot)
            mult(k, slot)
            start_scatter(k, slot)
            wait_scatter(k - 1, (1 + s) % 3)
            start_gather(k + 2, (1 + s) % 3)
        return 0

    lax.fori_loop(0, (NCH2 - 4) // 3, triple, 0)
    wait_gather(NCH2 - 2, (NCH2 - 2) % 3)
    mult(NCH2 - 2, (NCH2 - 2) % 3)
    start_scatter(NCH2 - 2, (NCH2 - 2) % 3)
    wait_gather(NCH2 - 1, (NCH2 - 1) % 3)
    mult(NCH2 - 1, (NCH2 - 1) % 3)
    start_scatter(NCH2 - 1, (NCH2 - 1) % 3)
    wait_scatter(NCH2 - 3, (NCH2 - 3) % 3)
    wait_scatter(NCH2 - 2, (NCH2 - 2) % 3)
    wait_scatter(NCH2 - 1, (NCH2 - 1) % 3)
    plsc.subcore_barrier()
    pltpu.sync_copy(acc_sh.at[pl.ds(sid * NPS, NPS)],
                    outp.at[cid, pl.ds(sid * NPS, NPS)])


@jax.jit
def _pass2(xs, rowi2, coli2s, u2):
    mesh = plsc.VectorSubcoreMesh(core_axis_name="c", subcore_axis_name="s")
    return pl.kernel(
        _pass2_body,
        out_type=jax.ShapeDtypeStruct((NC, NP_, DH), _f32),
        mesh=mesh,
        scratch_types=[
            pltpu.VMEM((NCH2, CH), _i32),
            pltpu.VMEM((NCH2, CH), _i32),
            pltpu.VMEM((NCH2, CH), _f32),
            pltpu.VMEM((CH, DH), _f32),
            pltpu.VMEM((CH, DH), _f32),
            pltpu.VMEM((CH, DH), _f32),
            pltpu.VMEM_SHARED((NP_, DH), _f32),
            pltpu.SemaphoreType.DMA,
            pltpu.SemaphoreType.DMA,
            pltpu.SemaphoreType.DMA,
            pltpu.SemaphoreType.DMA,
            pltpu.SemaphoreType.DMA,
            pltpu.SemaphoreType.DMA,
        ],
        compiler_params=pltpu.CompilerParams(needs_layout_passes=False,
                                             use_tc_tiling_on_sc=False),
    )(xs, rowi2, coli2s, u2)


BM = 1024


def _proj_body(x_ref, wr_ref, wc_ref, ar_ref, ac_ref):
    xv = x_ref[...]
    ar_ref[...] = jnp.dot(xv, wr_ref[...],
                          preferred_element_type=_f32).astype(_bf16)
    ac_ref[...] = jnp.dot(xv, wc_ref[...],
                          preferred_element_type=_f32).astype(_bf16)


BMP = 1000


@jax.jit
def _proj(x, wrt, wct):
    return pl.pallas_call(
        _proj_body,
        grid=(N // BMP,),
        in_specs=[
            pl.BlockSpec((BMP, DIN), lambda i: (i, 0)),
            pl.BlockSpec((DIN, DA), lambda i: (0, 0)),
            pl.BlockSpec((DIN, DA), lambda i: (0, 0)),
        ],
        out_specs=[
            pl.BlockSpec((BMP, DA), lambda i: (i, 0)),
            pl.BlockSpec((BMP, DA), lambda i: (i, 0)),
        ],
        out_shape=[
            jax.ShapeDtypeStruct((N, DA), _bf16),
            jax.ShapeDtypeStruct((N, DA), _bf16),
        ],
    )(x, wrt, wct)


def _final_body(p0_ref, p1_ref, w0_ref, w1_ref, s1t_ref, tpt_ref, b_ref,
                o_ref):
    mm = (jnp.dot(p0_ref[...], w0_ref[...], preferred_element_type=_f32)
          + jnp.dot(p1_ref[...], w1_ref[...], preferred_element_type=_f32))
    s = s1t_ref[...]
    t = tpt_ref[...]
    s1 = s[:, 0:1] + s[:, 1:2]
    tt = t[:, 0:1] + t[:, 1:2]
    d1 = 1.0 / (s1 + EPS)
    d2 = 1.0 / (d1 * tt + EPS)
    o_ref[...] = mm * (d1 * d2) + b_ref[...]


@jax.jit
def _final(p0, p1, wxt0, wxt1, s1pt, tpt, b):
    return pl.pallas_call(
        _final_body,
        grid=(NP_ // BM,),
        in_specs=[
            pl.BlockSpec((BM, DH), lambda i: (i, 0)),
            pl.BlockSpec((BM, DH), lambda i: (i, 0)),
            pl.BlockSpec((DH, DOUT), lambda i: (0, 0)),
            pl.BlockSpec((DH, DOUT), lambda i: (0, 0)),
            pl.BlockSpec((BM, 2), lambda i: (i, 0)),
            pl.BlockSpec((BM, 2), lambda i: (i, 0)),
            pl.BlockSpec((1, DOUT), lambda i: (0, 0)),
        ],
        out_specs=pl.BlockSpec((BM, DOUT), lambda i: (i, 0)),
        out_shape=jax.ShapeDtypeStruct((NP_, DOUT), _f32),
    )(p0, p1, wxt0, wxt1, s1pt, tpt, b)


def kernel(x, edge_index, adj_values, W_row, W_col, W_x, b_x):
    xs = jnp.concatenate([x[:, :DH], x[:, DH:]], axis=0)
    rowi2 = edge_index[0].reshape(NCHT, CH)
    coli2 = edge_index[1].reshape(NCHT, CH)
    coli2s = jnp.stack([coli2, coli2 + N])
    adj2 = adj_values.reshape(NCHT, CH)
    ar, ac = _proj(x, W_row.T, W_col.T)
    u2, s1p, tp = _pass1(ar, ac, rowi2, coli2, adj2)
    outp = _pass2(xs, rowi2, coli2s, u2)
    wxt = W_x.T
    out = _final(outp[0], outp[1], wxt[:DH], wxt[DH:],
                 s1p.T, tp.T, b_x.reshape(1, DOUT))
    return out[:N]


# speedup vs baseline: 1.4219x; 1.0581x over previous
import functools

import numpy as np

import jax
import jax.numpy as jnp
from jax import lax
from jax.experimental import pallas as pl
from jax.experimental.pallas import tpu as pltpu
from jax.experimental.pallas import tpu_sc as plsc

N = 10000
E = 320000
DIN = 128
DOUT = 128
DA = 64
NEG_SLOPE = 0.2
SHIFT = 8.0
EPS = 1e-15

NP_ = 10240
NC = 2
NS = 16
L = 16
NW = NC * NS
CH = 80
NCHT = E // CH
NCH = E // NW // CH
NPS = NP_ // NS
DH = DIN // 2
NCH2 = E // NS // CH

_f32 = jnp.float32
_bf16 = jnp.bfloat16
_i32 = jnp.int32


def _zero_1d(ref, n):
    def body(i, _):
        ref[pl.ds(i * L, L)] = jnp.zeros((L,), _f32)
        return 0
    lax.fori_loop(0, n // L, body, 0)


def _pass1_body(ar, ac, rowi2, coli2, adj2, u_out2, s1p, tp,
                idxr_v, idxc_v, adj_v, u_v, arr0, acr0, arr1, acr1,
                pbuf, pbuf2,
                s1t, tt, red_v, racc_v, sh, sg0, sg1):
    cid = lax.axis_index("c")
    sid = lax.axis_index("s")
    wid = sid * NC + cid
    base = wid * NCH

    _zero_1d(s1t, NP_)
    _zero_1d(tt, NP_)
    pltpu.sync_copy(rowi2.at[pl.ds(base, NCH)], idxr_v)
    pltpu.sync_copy(coli2.at[pl.ds(base, NCH)], idxc_v)
    pltpu.sync_copy(adj2.at[pl.ds(base, NCH)], adj_v)

    def start_g(k, arr, acr, sem):
        pltpu.async_copy(ar.at[idxr_v.at[k]], arr, sem)
        pltpu.async_copy(ac.at[idxc_v.at[k]], acr, sem)

    def wait_g(k, arr, acr, sem):
        pltpu.make_async_copy(ar.at[idxr_v.at[k]], arr, sem).wait()
        pltpu.make_async_copy(ac.at[idxc_v.at[k]], acr, sem).wait()

    iot = lax.iota(_i32, L) * L

    def compute(k, arr, acr):
        def stage(g, pb):
            for e in range(L):
                eg = g * L + e
                p0 = arr[eg, pl.ds(0, 2 * L)] * acr[eg, pl.ds(0, 2 * L)]
                p1 = (arr[eg, pl.ds(2 * L, 2 * L)]
                      * acr[eg, pl.ds(2 * L, 2 * L)])
                q0, q1 = plsc.unpack(p0, format=plsc.PackFormat.INTERLEAVED)
                q2, q3 = plsc.unpack(p1, format=plsc.PackFormat.INTERLEAVED)
                pb[pl.ds(e * L, L)] = (q0 + q1) + (q2 + q3)

        def finish(g, pb):
            vs = [plsc.load_gather(pb, [iot + j]) for j in range(L)]
            while len(vs) > 1:
                nxt = [vs[i] + vs[i + 1] for i in range(0, len(vs) - 1, 2)]
                if len(vs) % 2:
                    nxt.append(vs[-1])
                vs = nxt
            s = vs[0] * 0.125
            s = jnp.where(s >= 0, s, NEG_SLOPE * s)
            aa = jnp.exp(s - SHIFT)
            gl = pl.ds(g * L, L)
            uu = adj_v[k, gl] * aa
            u_v[k, gl] = uu
            ridx = idxr_v[k, gl]
            plsc.addupdate_scatter(s1t, [ridx], aa)
            plsc.addupdate_scatter(tt, [ridx], uu)

        bufs = (pbuf, pbuf2)
        stage(0, bufs[0])
        for g in range(1, CH // L):
            stage(g, bufs[g % 2])
            finish(g - 1, bufs[(g - 1) % 2])
        finish(CH // L - 1, bufs[(CH // L - 1) % 2])

    start_g(0, arr0, acr0, sg0)

    def pair(q, _):
        k = 2 * q
        start_g(k + 1, arr1, acr1, sg1)
        wait_g(k, arr0, acr0, sg0)
        compute(k, arr0, acr0)
        start_g(k + 2, arr0, acr0, sg0)
        wait_g(k + 1, arr1, acr1, sg1)
        compute(k + 1, arr1, acr1)
        return 0

    lax.fori_loop(0, NCH // 2, pair, 0)
    wait_g(NCH - 1, arr0, acr0, sg0)
    compute(NCH - 1, arr0, acr0)
    pltpu.sync_copy(u_v, u_out2.at[pl.ds(base, NCH)])

    pltpu.sync_copy(s1t, sh.at[sid, 0])
    pltpu.sync_copy(tt, sh.at[sid, 1])
    plsc.subcore_barrier()
    for which in (0, 1):
        _zero_1d(racc_v, NPS)

        def red_j(j, _):
            pltpu.sync_copy(sh.at[j, which, pl.ds(sid * NPS, NPS)], red_v)

            def addv(i, _):
                racc_v[pl.ds(i * L, L)] = (racc_v[pl.ds(i * L, L)]
                                           + red_v[pl.ds(i * L, L)])
                return 0

            lax.fori_loop(0, NPS // L, addv, 0)
            return 0

        lax.fori_loop(0, NS, red_j, 0)
        dst = s1p if which == 0 else tp
        pltpu.sync_copy(racc_v, dst.at[cid, pl.ds(sid * NPS, NPS)])


@jax.jit
def _pass1(ar, ac, rowi2, coli2, adj2):
    mesh = plsc.VectorSubcoreMesh(core_axis_name="c", subcore_axis_name="s")
    return pl.kernel(
        _pass1_body,
        out_type=(
            jax.ShapeDtypeStruct((NCHT, CH), _f32),
            jax.ShapeDtypeStruct((NC, NP_), _f32),
            jax.ShapeDtypeStruct((NC, NP_), _f32),
        ),
        mesh=mesh,
        scratch_types=[
            pltpu.VMEM((NCH, CH), _i32),
            pltpu.VMEM((NCH, CH), _i32),
            pltpu.VMEM((NCH, CH), _f32),
            pltpu.VMEM((NCH, CH), _f32),
            pltpu.VMEM((CH, DA), _bf16),
            pltpu.VMEM((CH, DA), _bf16),
            pltpu.VMEM((CH, DA), _bf16),
            pltpu.VMEM((CH, DA), _bf16),
            pltpu.VMEM((L * L,), _f32),
            pltpu.VMEM((L * L,), _f32),
            pltpu.VMEM((NP_,), _f32),
            pltpu.VMEM((NP_,), _f32),
            pltpu.VMEM((NPS,), _f32),
            pltpu.VMEM((NPS,), _f32),
            pltpu.VMEM_SHARED((NS, 2, NP_), _f32),
            pltpu.SemaphoreType.DMA,
            pltpu.SemaphoreType.DMA,
        ],
        compiler_params=pltpu.CompilerParams(needs_layout_passes=False,
                                             use_tc_tiling_on_sc=False),
    )(ar, ac, rowi2, coli2, adj2)


def _pass2_body(xs, rowi2, coli2s, u2, outp,
                idxr_v, idxc_v, uw_v,
                r0, r1, r2, f0, f1, f2, acc_sh,
                sg0, sg1, sg2, ss0, ss1, ss2):
    cid = lax.axis_index("c")
    sid = lax.axis_index("s")
    base = sid * NCH2

    pltpu.sync_copy(rowi2.at[pl.ds(base, NCH2)], idxr_v)
    pltpu.sync_copy(coli2s.at[cid, pl.ds(base, NCH2)], idxc_v)
    pltpu.sync_copy(u2.at[pl.ds(base, NCH2)], uw_v)

    def zrow(i, _):
        for j in range(DH // L):
            f0[i, pl.ds(j * L, L)] = jnp.zeros((L,), _f32)
        return 0

    lax.fori_loop(0, CH, zrow, 0)

    def zcopy(m, _):
        pltpu.sync_copy(f0, acc_sh.at[pl.ds(sid * NPS + m * CH, CH)])
        return 0

    lax.fori_loop(0, NPS // CH, zcopy, 0)
    plsc.subcore_barrier()

    rows = (r0, r1, r2)
    frows = (f0, f1, f2)
    sgs = (sg0, sg1, sg2)
    sss = (ss0, ss1, ss2)

    def start_gather(k, slot):
        pltpu.async_copy(xs.at[idxc_v.at[k]], rows[slot], sgs[slot])

    def wait_gather(k, slot):
        pltpu.make_async_copy(xs.at[idxc_v.at[k]], rows[slot],
                              sgs[slot]).wait()

    def start_scatter(k, slot):
        pltpu.async_copy(frows[slot], acc_sh.at[idxr_v.at[k]], sss[slot],
                         add=True)

    def wait_scatter(k, slot):
        pltpu.make_async_copy(frows[slot], acc_sh.at[idxr_v.at[k]],
                              sss[slot]).wait()

    def mult(k, slot):
        r = rows[slot]
        f = frows[slot]
        for g in range(CH // L):
            w16 = uw_v[k, pl.ds(g * L, L)]
            for e in range(L):
                w = w16[e]
                eg = g * L + e
                for h in range(DH // (2 * L)):
                    pk = r[eg, pl.ds(h * 2 * L, 2 * L)]
                    a, b = plsc.unpack(pk, format=plsc.PackFormat.INTERLEAVED)
                    f[eg, pl.ds(h * 2 * L, L)] = a * w
                    f[eg, pl.ds(h * 2 * L + L, L)] = b * w

    start_gather(0, 0)
    start_gather(1, 1)
    wait_gather(0, 0)
    mult(0, 0)
    start_scatter(0, 0)
    start_gather(2, 2)
    wait_gather(1, 1)
    mult(1, 1)
    start_scatter(1, 1)
    wait_scatter(0, 0)
    start_gather(3, 0)

    def triple(q, _):
        kq = 3 * q + 2
        for s in range(3):
            k = kq + s
            slot = (2 + s) % 3
            wait_gather(k, slot)
            mult(k, slot)
            start_scatter(k, slot)
            wait_scatter(k - 1, (1 + s) % 3)
            start_gather(k + 2, (1 + s) % 3)
        return 0

    lax.fori_loop(0, (NCH2 - 4) // 3, triple, 0)
    wait_gather(NCH2 - 2, (NCH2 - 2) % 3)
    mult(NCH2 - 2, (NCH2 - 2) % 3)
    start_scatter(NCH2 - 2, (NCH2 - 2) % 3)
    wait_gather(NCH2 - 1, (NCH2 - 1) % 3)
    mult(NCH2 - 1, (NCH2 - 1) % 3)
    start_scatter(NCH2 - 1, (NCH2 - 1) % 3)
    wait_scatter(NCH2 - 3, (NCH2 - 3) % 3)
    wait_scatter(NCH2 - 2, (NCH2 - 2) % 3)
    wait_scatter(NCH2 - 1, (NCH2 - 1) % 3)
    plsc.subcore_barrier()
    pltpu.sync_copy(acc_sh.at[pl.ds(sid * NPS, NPS)],
                    outp.at[cid, pl.ds(sid * NPS, NPS)])


@jax.jit
def _pass2(xs, rowi2, coli2s, u2):
    mesh = plsc.VectorSubcoreMesh(core_axis_name="c", subcore_axis_name="s")
    return pl.kernel(
        _pass2_body,
        out_type=jax.ShapeDtypeStruct((NC, NP_, DH), _f32),
        mesh=mesh,
        scratch_types=[
            pltpu.VMEM((NCH2, CH), _i32),
            pltpu.VMEM((NCH2, CH), _i32),
            pltpu.VMEM((NCH2, CH), _f32),
            pltpu.VMEM((CH, DH), _bf16),
            pltpu.VMEM((CH, DH), _bf16),
            pltpu.VMEM((CH, DH), _bf16),
            pltpu.VMEM((CH, DH), _f32),
            pltpu.VMEM((CH, DH), _f32),
            pltpu.VMEM((CH, DH), _f32),
            pltpu.VMEM_SHARED((NP_, DH), _f32),
            pltpu.SemaphoreType.DMA,
            pltpu.SemaphoreType.DMA,
            pltpu.SemaphoreType.DMA,
            pltpu.SemaphoreType.DMA,
            pltpu.SemaphoreType.DMA,
            pltpu.SemaphoreType.DMA,
        ],
        compiler_params=pltpu.CompilerParams(needs_layout_passes=False,
                                             use_tc_tiling_on_sc=False),
    )(xs, rowi2, coli2s, u2)


BM = 1024


def _proj_body(x_ref, wr_ref, wc_ref, ar_ref, ac_ref):
    xv = x_ref[...]
    ar_ref[...] = jnp.dot(xv, wr_ref[...],
                          preferred_element_type=_f32).astype(_bf16)
    ac_ref[...] = jnp.dot(xv, wc_ref[...],
                          preferred_element_type=_f32).astype(_bf16)


BMP = 1000


@jax.jit
def _proj(x, wrt, wct):
    return pl.pallas_call(
        _proj_body,
        grid=(N // BMP,),
        in_specs=[
            pl.BlockSpec((BMP, DIN), lambda i: (i, 0)),
            pl.BlockSpec((DIN, DA), lambda i: (0, 0)),
            pl.BlockSpec((DIN, DA), lambda i: (0, 0)),
        ],
        out_specs=[
            pl.BlockSpec((BMP, DA), lambda i: (i, 0)),
            pl.BlockSpec((BMP, DA), lambda i: (i, 0)),
        ],
        out_shape=[
            jax.ShapeDtypeStruct((N, DA), _bf16),
            jax.ShapeDtypeStruct((N, DA), _bf16),
        ],
    )(x, wrt, wct)


def _final_body(p0_ref, p1_ref, w0_ref, w1_ref, s1t_ref, tpt_ref, b_ref,
                o_ref):
    mm = (jnp.dot(p0_ref[...], w0_ref[...], preferred_element_type=_f32)
          + jnp.dot(p1_ref[...], w1_ref[...], preferred_element_type=_f32))
    s = s1t_ref[...]
    t = tpt_ref[...]
    s1 = s[:, 0:1] + s[:, 1:2]
    tt = t[:, 0:1] + t[:, 1:2]
    d1 = 1.0 / (s1 + EPS)
    d2 = 1.0 / (d1 * tt + EPS)
    o_ref[...] = mm * (d1 * d2) + b_ref[...]


@jax.jit
def _final(p0, p1, wxt0, wxt1, s1pt, tpt, b):
    return pl.pallas_call(
        _final_body,
        grid=(NP_ // BM,),
        in_specs=[
            pl.BlockSpec((BM, DH), lambda i: (i, 0)),
            pl.BlockSpec((BM, DH), lambda i: (i, 0)),
            pl.BlockSpec((DH, DOUT), lambda i: (0, 0)),
            pl.BlockSpec((DH, DOUT), lambda i: (0, 0)),
            pl.BlockSpec((BM, 2), lambda i: (i, 0)),
            pl.BlockSpec((BM, 2), lambda i: (i, 0)),
            pl.BlockSpec((1, DOUT), lambda i: (0, 0)),
        ],
        out_specs=pl.BlockSpec((BM, DOUT), lambda i: (i, 0)),
        out_shape=jax.ShapeDtypeStruct((NP_, DOUT), _f32),
    )(p0, p1, wxt0, wxt1, s1pt, tpt, b)


_PERM = np.concatenate(
    [np.concatenate([np.arange(0, 2 * L, 2), np.arange(1, 2 * L, 2)])
     + g * 2 * L for g in range(DH // (2 * L))])


def kernel(x, edge_index, adj_values, W_row, W_col, W_x, b_x):
    xs = jnp.concatenate([x[:, :DH], x[:, DH:]],
                         axis=0).astype(_bf16)
    rowi2 = edge_index[0].reshape(NCHT, CH)
    coli2 = edge_index[1].reshape(NCHT, CH)
    coli2s = jnp.stack([coli2, coli2 + N])
    adj2 = adj_values.reshape(NCHT, CH)
    ar, ac = _proj(x, W_row.T, W_col.T)
    u2, s1p, tp = _pass1(ar, ac, rowi2, coli2, adj2)
    outp = _pass2(xs, rowi2, coli2s, u2)
    wxt = W_x.T
    out = _final(outp[0], outp[1], wxt[:DH][_PERM], wxt[DH:][_PERM],
                 s1p.T, tp.T, b_x.reshape(1, DOUT))
    return out[:N]


# speedup vs baseline: 1.4227x; 1.0006x over previous
import numpy as np

import jax
import jax.numpy as jnp
from jax import lax
from jax.experimental import pallas as pl
from jax.experimental.pallas import tpu as pltpu
from jax.experimental.pallas import tpu_sc as plsc

N = 10000
E = 320000
DIN = 128
DOUT = 128
DA = 64
NEG_SLOPE = 0.2
SHIFT = 8.0
EPS = 1e-15

NP_ = 10240
NC = 2
NS = 16
L = 16
NW = NC * NS
CH = 80
NCHT = E // CH
NCH = E // NW // CH
NPS = NP_ // NS
DH = DIN // 2
NCH2 = E // NS // CH

_f32 = jnp.float32
_bf16 = jnp.bfloat16
_i32 = jnp.int32


def _zero_1d(ref, n):
    def body(i, _):
        ref[pl.ds(i * L, L)] = jnp.zeros((L,), _f32)
        return 0
    lax.fori_loop(0, n // L, body, 0)


def _pass1_body(ar, ac, rowi2, coli2, adj2, u_out2, s1p, tp,
                idxr_v, idxc_v, adj_v, u_v, arr0, acr0, arr1, acr1,
                pbuf, pbuf2,
                s1t, tt, red_v, racc_v, sh, sg0, sg1):
    cid = lax.axis_index("c")
    sid = lax.axis_index("s")
    wid = sid * NC + cid
    base = wid * NCH

    _zero_1d(s1t, NP_)
    _zero_1d(tt, NP_)
    pltpu.sync_copy(rowi2.at[pl.ds(base, NCH)], idxr_v)
    pltpu.sync_copy(coli2.at[pl.ds(base, NCH)], idxc_v)
    pltpu.sync_copy(adj2.at[pl.ds(base, NCH)], adj_v)

    def start_g(k, arr, acr, sem):
        pltpu.async_copy(ar.at[idxr_v.at[k]], arr, sem)
        pltpu.async_copy(ac.at[idxc_v.at[k]], acr, sem)

    def wait_g(k, arr, acr, sem):
        pltpu.make_async_copy(ar.at[idxr_v.at[k]], arr, sem).wait()
        pltpu.make_async_copy(ac.at[idxc_v.at[k]], acr, sem).wait()

    iot = lax.iota(_i32, L) * L

    def compute(k, arr, acr):
        def stage(g, pb):
            for e in range(L):
                eg = g * L + e
                p0 = arr[eg, pl.ds(0, 2 * L)] * acr[eg, pl.ds(0, 2 * L)]
                p1 = (arr[eg, pl.ds(2 * L, 2 * L)]
                      * acr[eg, pl.ds(2 * L, 2 * L)])
                q0, q1 = plsc.unpack(p0, format=plsc.PackFormat.INTERLEAVED)
                q2, q3 = plsc.unpack(p1, format=plsc.PackFormat.INTERLEAVED)
                pb[pl.ds(e * L, L)] = (q0 + q1) + (q2 + q3)

        def finish(g, pb):
            vs = [plsc.load_gather(pb, [iot + j]) for j in range(L)]
            while len(vs) > 1:
                nxt = [vs[i] + vs[i + 1] for i in range(0, len(vs) - 1, 2)]
                if len(vs) % 2:
                    nxt.append(vs[-1])
                vs = nxt
            s = vs[0] * 0.125
            s = jnp.where(s >= 0, s, NEG_SLOPE * s)
            aa = jnp.exp(s - SHIFT)
            gl = pl.ds(g * L, L)
            uu = adj_v[k, gl] * aa
            u_v[k, gl] = uu
            ridx = idxr_v[k, gl]
            plsc.addupdate_scatter(s1t, [ridx], aa)
            plsc.addupdate_scatter(tt, [ridx], uu)

        bufs = (pbuf, pbuf2)
        stage(0, bufs[0])
        for g in range(1, CH // L):
            stage(g, bufs[g % 2])
            finish(g - 1, bufs[(g - 1) % 2])
        finish(CH // L - 1, bufs[(CH // L - 1) % 2])

    start_g(0, arr0, acr0, sg0)

    def pair(q, _):
        k = 2 * q
        start_g(k + 1, arr1, acr1, sg1)
        wait_g(k, arr0, acr0, sg0)
        compute(k, arr0, acr0)
        start_g(k + 2, arr0, acr0, sg0)
        wait_g(k + 1, arr1, acr1, sg1)
        compute(k + 1, arr1, acr1)
        return 0

    lax.fori_loop(0, NCH // 2, pair, 0)
    wait_g(NCH - 1, arr0, acr0, sg0)
    compute(NCH - 1, arr0, acr0)
    pltpu.sync_copy(u_v, u_out2.at[pl.ds(base, NCH)])

    pltpu.sync_copy(s1t, sh.at[sid, 0])
    pltpu.sync_copy(tt, sh.at[sid, 1])
    plsc.subcore_barrier()
    for which in (0, 1):
        _zero_1d(racc_v, NPS)

        def red_j(j, _):
            pltpu.sync_copy(sh.at[j, which, pl.ds(sid * NPS, NPS)], red_v)

            def addv(i, _):
                racc_v[pl.ds(i * L, L)] = (racc_v[pl.ds(i * L, L)]
                                           + red_v[pl.ds(i * L, L)])
                return 0

            lax.fori_loop(0, NPS // L, addv, 0)
            return 0

        lax.fori_loop(0, NS, red_j, 0)
        dst = s1p if which == 0 else tp
        pltpu.sync_copy(racc_v, dst.at[cid, pl.ds(sid * NPS, NPS)])


@jax.jit
def _pass1(ar, ac, rowi2, coli2, adj2):
    mesh = plsc.VectorSubcoreMesh(core_axis_name="c", subcore_axis_name="s")
    return pl.kernel(
        _pass1_body,
        out_type=(
            jax.ShapeDtypeStruct((NCHT, CH), _f32),
            jax.ShapeDtypeStruct((NC, NP_), _f32),
            jax.ShapeDtypeStruct((NC, NP_), _f32),
        ),
        mesh=mesh,
        scratch_types=[
            pltpu.VMEM((NCH, CH), _i32),
            pltpu.VMEM((NCH, CH), _i32),
            pltpu.VMEM((NCH, CH), _f32),
            pltpu.VMEM((NCH, CH), _f32),
            pltpu.VMEM((CH, DA), _bf16),
            pltpu.VMEM((CH, DA), _bf16),
            pltpu.VMEM((CH, DA), _bf16),
            pltpu.VMEM((CH, DA), _bf16),
            pltpu.VMEM((L * L,), _f32),
            pltpu.VMEM((L * L,), _f32),
            pltpu.VMEM((NP_,), _f32),
            pltpu.VMEM((NP_,), _f32),
            pltpu.VMEM((NPS,), _f32),
            pltpu.VMEM((NPS,), _f32),
            pltpu.VMEM_SHARED((NS, 2, NP_), _f32),
            pltpu.SemaphoreType.DMA,
            pltpu.SemaphoreType.DMA,
        ],
        compiler_params=pltpu.CompilerParams(needs_layout_passes=False,
                                             use_tc_tiling_on_sc=False),
    )(ar, ac, rowi2, coli2, adj2)


def _pass2_body(xs, rowi2, coli2s, u2, outp,
                idxr_v, idxc_v, uw_v,
                r0, r1, r2, f0, f1, f2, acc_sh,
                sg0, sg1, sg2, ss0, ss1, ss2):
    cid = lax.axis_index("c")
    sid = lax.axis_index("s")
    base = sid * NCH2

    pltpu.sync_copy(rowi2.at[pl.ds(base, NCH2)], idxr_v)
    pltpu.sync_copy(coli2s.at[cid, pl.ds(base, NCH2)], idxc_v)
    pltpu.sync_copy(u2.at[pl.ds(base, NCH2)], uw_v)

    def zrow(i, _):
        for j in range(DH // L):
            f0[i, pl.ds(j * L, L)] = jnp.zeros((L,), _f32)
        return 0

    lax.fori_loop(0, CH, zrow, 0)

    def zcopy(m, _):
        pltpu.sync_copy(f0, acc_sh.at[pl.ds(sid * NPS + m * CH, CH)])
        return 0

    lax.fori_loop(0, NPS // CH, zcopy, 0)
    plsc.subcore_barrier()

    rows = (r0, r1, r2)
    frows = (f0, f1, f2)
    sgs = (sg0, sg1, sg2)
    sss = (ss0, ss1, ss2)

    def start_gather(k, slot):
        pltpu.async_copy(xs.at[idxc_v.at[k]], rows[slot], sgs[slot])

    def wait_gather(k, slot):
        pltpu.make_async_copy(xs.at[idxc_v.at[k]], rows[slot],
                              sgs[slot]).wait()

    def start_scatter(k, slot):
        pltpu.async_copy(frows[slot], acc_sh.at[idxr_v.at[k]], sss[slot],
                         add=True)

    def wait_scatter(k, slot):
        pltpu.make_async_copy(frows[slot], acc_sh.at[idxr_v.at[k]],
                              sss[slot]).wait()

    def mult(k, slot):
        r = rows[slot]
        f = frows[slot]
        for g in range(CH // L):
            w16 = uw_v[k, pl.ds(g * L, L)]
            for e in range(L):
                w = w16[e]
                eg = g * L + e
                for h in range(DH // (2 * L)):
                    pk = r[eg, pl.ds(h * 2 * L, 2 * L)]
                    a, b = plsc.unpack(pk, format=plsc.PackFormat.INTERLEAVED)
                    f[eg, pl.ds(h * 2 * L, L)] = a * w
                    f[eg, pl.ds(h * 2 * L + L, L)] = b * w

    start_gather(0, 0)
    start_gather(1, 1)
    wait_gather(0, 0)
    mult(0, 0)
    start_scatter(0, 0)
    start_gather(2, 2)
    wait_gather(1, 1)
    mult(1, 1)
    start_scatter(1, 1)
    wait_scatter(0, 0)
    start_gather(3, 0)

    def triple(q, _):
        kq = 3 * q + 2
        for s in range(3):
            k = kq + s
            slot = (2 + s) % 3
            wait_gather(k, slot)
            mult(k, slot)
            start_scatter(k, slot)
            wait_scatter(k - 1, (1 + s) % 3)
            start_gather(k + 2, (1 + s) % 3)
        return 0

    lax.fori_loop(0, (NCH2 - 4) // 3, triple, 0)
    wait_gather(NCH2 - 2, (NCH2 - 2) % 3)
    mult(NCH2 - 2, (NCH2 - 2) % 3)
    start_scatter(NCH2 - 2, (NCH2 - 2) % 3)
    wait_gather(NCH2 - 1, (NCH2 - 1) % 3)
    mult(NCH2 - 1, (NCH2 - 1) % 3)
    start_scatter(NCH2 - 1, (NCH2 - 1) % 3)
    wait_scatter(NCH2 - 3, (NCH2 - 3) % 3)
    wait_scatter(NCH2 - 2, (NCH2 - 2) % 3)
    wait_scatter(NCH2 - 1, (NCH2 - 1) % 3)
    plsc.subcore_barrier()
    pltpu.sync_copy(acc_sh.at[pl.ds(sid * NPS, NPS)],
                    outp.at[cid, pl.ds(sid * NPS, NPS)])


@jax.jit
def _pass2(xs, rowi2, coli2s, u2):
    mesh = plsc.VectorSubcoreMesh(core_axis_name="c", subcore_axis_name="s")
    return pl.kernel(
        _pass2_body,
        out_type=jax.ShapeDtypeStruct((NC, NP_, DH), _f32),
        mesh=mesh,
        scratch_types=[
            pltpu.VMEM((NCH2, CH), _i32),
            pltpu.VMEM((NCH2, CH), _i32),
            pltpu.VMEM((NCH2, CH), _f32),
            pltpu.VMEM((CH, DH), _bf16),
            pltpu.VMEM((CH, DH), _bf16),
            pltpu.VMEM((CH, DH), _bf16),
            pltpu.VMEM((CH, DH), _f32),
            pltpu.VMEM((CH, DH), _f32),
            pltpu.VMEM((CH, DH), _f32),
            pltpu.VMEM_SHARED((NP_, DH), _f32),
            pltpu.SemaphoreType.DMA,
            pltpu.SemaphoreType.DMA,
            pltpu.SemaphoreType.DMA,
            pltpu.SemaphoreType.DMA,
            pltpu.SemaphoreType.DMA,
            pltpu.SemaphoreType.DMA,
        ],
        compiler_params=pltpu.CompilerParams(needs_layout_passes=False,
                                             use_tc_tiling_on_sc=False),
    )(xs, rowi2, coli2s, u2)


BM = 1024


def _proj_body(x_ref, wr_ref, wc_ref, ar_ref, ac_ref):
    xv = x_ref[...]
    ar_ref[...] = jnp.dot(xv, wr_ref[...],
                          preferred_element_type=_f32).astype(_bf16)
    ac_ref[...] = jnp.dot(xv, wc_ref[...],
                          preferred_element_type=_f32).astype(_bf16)


BMP = 1000


@jax.jit
def _proj(x, wrt, wct):
    return pl.pallas_call(
        _proj_body,
        grid=(N // BMP,),
        in_specs=[
            pl.BlockSpec((BMP, DIN), lambda i: (i, 0)),
            pl.BlockSpec((DIN, DA), lambda i: (0, 0)),
            pl.BlockSpec((DIN, DA), lambda i: (0, 0)),
        ],
        out_specs=[
            pl.BlockSpec((BMP, DA), lambda i: (i, 0)),
            pl.BlockSpec((BMP, DA), lambda i: (i, 0)),
        ],
        out_shape=[
            jax.ShapeDtypeStruct((N, DA), _bf16),
            jax.ShapeDtypeStruct((N, DA), _bf16),
        ],
    )(x, wrt, wct)


def _final_body(p0_ref, p1_ref, w0_ref, w1_ref, s1t_ref, tpt_ref, b_ref,
                o_ref):
    mm = (jnp.dot(p0_ref[...], w0_ref[...], preferred_element_type=_f32)
          + jnp.dot(p1_ref[...], w1_ref[...], preferred_element_type=_f32))
    s = s1t_ref[...]
    t = tpt_ref[...]
    s1 = s[:, 0:1] + s[:, 1:2]
    tt = t[:, 0:1] + t[:, 1:2]
    d1 = 1.0 / (s1 + EPS)
    d2 = 1.0 / (d1 * tt + EPS)
    o_ref[...] = mm * (d1 * d2) + b_ref[...]


@jax.jit
def _final(p0, p1, wxt0, wxt1, s1pt, tpt, b):
    return pl.pallas_call(
        _final_body,
        grid=(NP_ // BM,),
        in_specs=[
            pl.BlockSpec((BM, DH), lambda i: (i, 0)),
            pl.BlockSpec((BM, DH), lambda i: (i, 0)),
            pl.BlockSpec((DH, DOUT), lambda i: (0, 0)),
            pl.BlockSpec((DH, DOUT), lambda i: (0, 0)),
            pl.BlockSpec((BM, 2), lambda i: (i, 0)),
            pl.BlockSpec((BM, 2), lambda i: (i, 0)),
            pl.BlockSpec((1, DOUT), lambda i: (0, 0)),
        ],
        out_specs=pl.BlockSpec((BM, DOUT), lambda i: (i, 0)),
        out_shape=jax.ShapeDtypeStruct((NP_, DOUT), _f32),
    )(p0, p1, wxt0, wxt1, s1pt, tpt, b)


_PERM = np.concatenate(
    [np.concatenate([np.arange(0, 2 * L, 2), np.arange(1, 2 * L, 2)])
     + g * 2 * L for g in range(DH // (2 * L))])


def kernel(x, edge_index, adj_values, W_row, W_col, W_x, b_x):
    xs = jnp.concatenate([x[:, :DH], x[:, DH:]],
                         axis=0).astype(_bf16)
    rowi2 = edge_index[0].reshape(NCHT, CH)
    coli2 = edge_index[1].reshape(NCHT, CH)
    coli2s = jnp.stack([coli2, coli2 + N])
    adj2 = adj_values.reshape(NCHT, CH)
    ar, ac = _proj(x, W_row.T, W_col.T)
    u2, s1p, tp = _pass1(ar, ac, rowi2, coli2, adj2)
    outp = _pass2(xs, rowi2, coli2s, u2)
    wxt = W_x.T
    out = _final(outp[0], outp[1], wxt[:DH][_PERM], wxt[DH:][_PERM],
                 s1p.T, tp.T, b_x.reshape(1, DOUT))
    return out[:N]
